# Initial kernel scaffold; baseline (speedup 1.0000x reference)
import functools

import jax
import jax.numpy as jnp
from jax import lax
from jax.experimental import pallas as pl
from jax.experimental.pallas import tpu as pltpu
from jax.experimental.pallas import tpu_sc as plsc

N = 10000
E = 320000
D_IN = 128
EMB = 48
K = 7
D_OUT = 10

NC = 2
NS = 16
NW = NC * NS
LANES = 16

Np = 10240
Ep = 327680
TE = Ep // NW
CHUNK = 512
RI = CHUNK // 128
NCHUNK = TE // CHUNK
TROW = Np // NS

_mesh = plsc.VectorSubcoreMesh(core_axis_name="c", subcore_axis_name="s",
                               num_cores=NC, num_subcores=NS)


def _deg_body(row2d, ew2d, zn, p0, p1, acc, idx_v, ew_v):
    c_id = lax.axis_index("c")
    s_id = lax.axis_index("s")
    wid = c_id * NS + s_id
    pltpu.sync_copy(zn.at[pl.ds(s_id * TROW, TROW)],
                    acc.at[pl.ds(s_id * TROW, TROW)])
    plsc.subcore_barrier()
    base_row = wid * (TE // 128)

    def chunk(ci, carry):
        rbase = base_row + ci * RI
        pltpu.sync_copy(row2d.at[pl.ds(rbase, RI)], idx_v)
        pltpu.sync_copy(ew2d.at[pl.ds(rbase, RI)], ew_v)
        for j in range(RI):
            pltpu.sync_copy(ew_v.at[j], acc.at[idx_v.at[j]], add=True)
        return carry

    lax.fori_loop(0, NCHUNK, chunk, 0)
    plsc.subcore_barrier()

    @pl.when(c_id == 0)
    def _():
        pltpu.sync_copy(acc.at[pl.ds(s_id * TROW, TROW)],
                        p0.at[pl.ds(s_id * TROW, TROW)])

    @pl.when(c_id == 1)
    def _():
        pltpu.sync_copy(acc.at[pl.ds(s_id * TROW, TROW)],
                        p1.at[pl.ds(s_id * TROW, TROW)])


_deg_kernel = pl.kernel(
    _deg_body,
    out_type=(jax.ShapeDtypeStruct((Np,), jnp.float32),
              jax.ShapeDtypeStruct((Np,), jnp.float32)),
    mesh=_mesh,
    scratch_types=[
        pltpu.VMEM_SHARED((Np,), jnp.float32),
        pltpu.VMEM((RI, 128), jnp.int32),
        pltpu.VMEM((RI, 128), jnp.float32),
    ],
)


def _norm_body(p0, p1, rowf, colf, ewf, norm_out,
               dis_v, tmp_v, row_v, col_v, ew_v, nrm_v):
    c_id = lax.axis_index("c")
    s_id = lax.axis_index("s")
    wid = c_id * NS + s_id
    pltpu.sync_copy(p0, dis_v)
    pltpu.sync_copy(p1, tmp_v)

    def dloop(i, carry):
        d = dis_v[pl.ds(i * 16, 16)] + tmp_v[pl.ds(i * 16, 16)]
        bits = plsc.bitcast(d, jnp.int32)
        y = plsc.bitcast(jnp.int32(0x5F3759DF) - (bits >> 1), jnp.float32)
        y = y * (1.5 - 0.5 * d * y * y)
        y = y * (1.5 - 0.5 * d * y * y)
        y = y * (1.5 - 0.5 * d * y * y)
        dis_v[pl.ds(i * 16, 16)] = jnp.where(d > 0.0, y, 0.0)
        return carry

    lax.fori_loop(0, Np // 16, dloop, 0)

    base = wid * TE
    pltpu.sync_copy(rowf.at[pl.ds(base, TE)], row_v)
    pltpu.sync_copy(colf.at[pl.ds(base, TE)], col_v)
    pltpu.sync_copy(ewf.at[pl.ds(base, TE)], ew_v)

    def eloop(i, carry):
        r16 = row_v[pl.ds(i * 16, 16)]
        c16 = col_v[pl.ds(i * 16, 16)]
        a = plsc.load_gather(dis_v, [r16])
        b = plsc.load_gather(dis_v, [c16])
        nrm_v[pl.ds(i * 16, 16)] = -(a * ew_v[pl.ds(i * 16, 16)] * b)
        return carry

    lax.fori_loop(0, TE // 16, eloop, 0)
    pltpu.sync_copy(nrm_v, norm_out.at[pl.ds(base, TE)])


_norm_kernel = pl.kernel(
    _norm_body,
    out_type=jax.ShapeDtypeStruct((Ep,), jnp.float32),
    mesh=_mesh,
    scratch_types=[
        pltpu.VMEM((Np,), jnp.float32),
        pltpu.VMEM((Np,), jnp.float32),
        pltpu.VMEM((TE,), jnp.int32),
        pltpu.VMEM((TE,), jnp.int32),
        pltpu.VMEM((TE,), jnp.float32),
        pltpu.VMEM((TE,), jnp.float32),
    ],
)


def _hop_body(t_hbm, row2d, col2d, nrmf, z48, p0, p1,
              acc, rowi_v, coli_v, nrm_v, rows_v, sem):
    c_id = lax.axis_index("c")
    s_id = lax.axis_index("s")
    wid = c_id * NS + s_id
    pltpu.sync_copy(z48.at[pl.ds(s_id * TROW, TROW)],
                    acc.at[pl.ds(s_id * TROW, TROW)])
    plsc.subcore_barrier()
    base_row = wid * (TE // 128)
    ebase = wid * TE

    def chunk(ci, carry):
        rbase = base_row + ci * RI
        pltpu.sync_copy(row2d.at[pl.ds(rbase, RI)], rowi_v)
        pltpu.sync_copy(col2d.at[pl.ds(rbase, RI)], coli_v)
        pltpu.sync_copy(nrmf.at[pl.ds(ebase + ci * CHUNK, CHUNK)], nrm_v)
        descs = [
            pltpu.async_copy(t_hbm.at[rowi_v.at[j]],
                             rows_v.at[pl.ds(j * 128, 128)], sem)
            for j in range(RI)
        ]
        for d in descs:
            d.wait()

        def scale(e, c2):
            s = nrm_v[e]
            for j in range(3):
                rows_v[e, pl.ds(j * 16, 16)] = rows_v[e, pl.ds(j * 16, 16)] * s
            return c2

        lax.fori_loop(0, CHUNK, scale, 0)
        for j in range(RI):
            pltpu.sync_copy(rows_v.at[pl.ds(j * 128, 128)],
                            acc.at[coli_v.at[j]], add=True)
        return carry

    lax.fori_loop(0, NCHUNK, chunk, 0)
    plsc.subcore_barrier()

    @pl.when(c_id == 0)
    def _():
        pltpu.sync_copy(acc.at[pl.ds(s_id * TROW, TROW)],
                        p0.at[pl.ds(s_id * TROW, TROW)])

    @pl.when(c_id == 1)
    def _():
        pltpu.sync_copy(acc.at[pl.ds(s_id * TROW, TROW)],
                        p1.at[pl.ds(s_id * TROW, TROW)])


_hop_kernel = pl.kernel(
    _hop_body,
    out_type=(jax.ShapeDtypeStruct((Np, EMB), jnp.float32),
              jax.ShapeDtypeStruct((Np, EMB), jnp.float32)),
    mesh=_mesh,
    scratch_types=[
        pltpu.VMEM_SHARED((Np, EMB), jnp.float32),
        pltpu.VMEM((RI, 128), jnp.int32),
        pltpu.VMEM((RI, 128), jnp.int32),
        pltpu.VMEM((CHUNK,), jnp.float32),
        pltpu.VMEM((CHUNK, EMB), jnp.float32),
        pltpu.SemaphoreType.DMA,
    ],
)


_BLK = 1000
_NBLK = N // _BLK


def _input_body(x_ref, w_ref, b_ref, o_ref):
    h = jnp.dot(x_ref[...], w_ref[...], preferred_element_type=jnp.float32)
    o_ref[...] = jnp.maximum(h + b_ref[...], 0.0)


_input_kernel = pl.pallas_call(
    _input_body,
    grid=(_NBLK,),
    in_specs=[
        pl.BlockSpec((_BLK, D_IN), lambda i: (i, 0)),
        pl.BlockSpec((D_IN, EMB), lambda i: (0, 0)),
        pl.BlockSpec((1, EMB), lambda i: (0, 0)),
    ],
    out_specs=pl.BlockSpec((_BLK, EMB), lambda i: (i, 0)),
    out_shape=jax.ShapeDtypeStruct((N, EMB), jnp.float32),
)


def _comb1_body(p0_ref, p1_ref, o_ref):
    o_ref[...] = p0_ref[...] + p1_ref[...]


_comb1_kernel = pl.pallas_call(
    _comb1_body,
    grid=(_NBLK,),
    in_specs=[
        pl.BlockSpec((_BLK, EMB), lambda i: (i, 0)),
        pl.BlockSpec((_BLK, EMB), lambda i: (i, 0)),
    ],
    out_specs=pl.BlockSpec((_BLK, EMB), lambda i: (i, 0)),
    out_shape=jax.ShapeDtypeStruct((N, EMB), jnp.float32),
)


def _comb_body(p0_ref, p1_ref, tm2_ref, o_ref):
    o_ref[...] = 2.0 * (p0_ref[...] + p1_ref[...]) - tm2_ref[...]


_comb_kernel = pl.pallas_call(
    _comb_body,
    grid=(_NBLK,),
    in_specs=[
        pl.BlockSpec((_BLK, EMB), lambda i: (i, 0)),
        pl.BlockSpec((_BLK, EMB), lambda i: (i, 0)),
        pl.BlockSpec((_BLK, EMB), lambda i: (i, 0)),
    ],
    out_specs=pl.BlockSpec((_BLK, EMB), lambda i: (i, 0)),
    out_shape=jax.ShapeDtypeStruct((N, EMB), jnp.float32),
)


def _final_body(t0, t1, t2, t3, t4, t5, t6, cw, cb, wo, bo, o_ref):
    ts = (t0, t1, t2, t3, t4, t5, t6)
    acc = jnp.dot(ts[0][...], cw[0], preferred_element_type=jnp.float32)
    for k in range(1, K):
        acc = acc + jnp.dot(ts[k][...], cw[k],
                            preferred_element_type=jnp.float32)
    acc = jnp.maximum(acc + cb[...], 0.0)
    o_ref[...] = jnp.dot(acc, wo[...],
                         preferred_element_type=jnp.float32) + bo[...]


_final_kernel = pl.pallas_call(
    _final_body,
    grid=(_NBLK,),
    in_specs=[pl.BlockSpec((_BLK, EMB), lambda i: (i, 0))] * K + [
        pl.BlockSpec((K, EMB, EMB), lambda i: (0, 0, 0)),
        pl.BlockSpec((1, EMB), lambda i: (0, 0)),
        pl.BlockSpec((EMB, D_OUT), lambda i: (0, 0)),
        pl.BlockSpec((1, D_OUT), lambda i: (0, 0)),
    ],
    out_specs=pl.BlockSpec((_BLK, D_OUT), lambda i: (i, 0)),
    out_shape=jax.ShapeDtypeStruct((N, D_OUT), jnp.float32),
)


def kernel(x, edge_index, edge_weight, W_in, b_in, cheb_W, cheb_b, W_out,
           b_out):
    row = edge_index[0].astype(jnp.int32)
    col = edge_index[1].astype(jnp.int32)
    pad = Ep - E
    rowp = jnp.concatenate([row, jnp.zeros((pad,), jnp.int32)])
    colp = jnp.concatenate([col, jnp.zeros((pad,), jnp.int32)])
    ewp = jnp.concatenate([edge_weight, jnp.zeros((pad,), jnp.float32)])
    row2d = rowp.reshape(Ep // 128, 128)
    col2d = colp.reshape(Ep // 128, 128)
    ew2d = ewp.reshape(Ep // 128, 128)
    zn = jnp.zeros((Np,), jnp.float32)
    z48 = jnp.zeros((Np, EMB), jnp.float32)

    dp0, dp1 = _deg_kernel(row2d, ew2d, zn)
    norm = _norm_kernel(dp0, dp1, rowp, colp, ewp)
    h = _input_kernel(x, W_in, b_in.reshape(1, EMB))

    ts = [h]
    p0, p1 = _hop_kernel(h, row2d, col2d, norm, z48)
    t1 = _comb1_kernel(p0, p1)
    ts.append(t1)
    tkm2, tkm1 = h, t1
    for _ in range(2, K):
        p0, p1 = _hop_kernel(tkm1, row2d, col2d, norm, z48)
        tk = _comb_kernel(p0, p1, tkm2)
        ts.append(tk)
        tkm2, tkm1 = tkm1, tk

    out = _final_kernel(ts[0], ts[1], ts[2], ts[3], ts[4], ts[5], ts[6],
                        cheb_W, cheb_b.reshape(1, EMB), W_out,
                        b_out.reshape(1, D_OUT))
    return (out, h)


# speedup vs baseline: 8.3143x; 8.3143x over previous
import functools

import jax
import jax.numpy as jnp
from jax import lax
from jax.experimental import pallas as pl
from jax.experimental.pallas import tpu as pltpu
from jax.experimental.pallas import tpu_sc as plsc

N = 10000
E = 320000
D_IN = 128
EMB = 48
K = 7
D_OUT = 10

NC = 2
NS = 16
NW = NC * NS
LANES = 16

Np = 10240
Ep = 327680
TE = Ep // NW
CHUNK = 512
RI = CHUNK // 128
NCHUNK = TE // CHUNK
TROW = Np // NS

_mesh = plsc.VectorSubcoreMesh(core_axis_name="c", subcore_axis_name="s",
                               num_cores=NC, num_subcores=NS)


def _deg_body(row2d, ew2d, zn, p0, p1, acc, idx_v, ew_v):
    c_id = lax.axis_index("c")
    s_id = lax.axis_index("s")
    wid = c_id * NS + s_id
    pltpu.sync_copy(zn.at[pl.ds(s_id * TROW, TROW)],
                    acc.at[pl.ds(s_id * TROW, TROW)])
    plsc.subcore_barrier()
    base_row = wid * (TE // 128)

    def chunk(ci, carry):
        rbase = base_row + ci * RI
        pltpu.sync_copy(row2d.at[pl.ds(rbase, RI)], idx_v)
        pltpu.sync_copy(ew2d.at[pl.ds(rbase, RI)], ew_v)
        for j in range(RI):
            pltpu.sync_copy(ew_v.at[j], acc.at[idx_v.at[j]], add=True)
        return carry

    lax.fori_loop(0, NCHUNK, chunk, 0)
    plsc.subcore_barrier()

    @pl.when(c_id == 0)
    def _():
        pltpu.sync_copy(acc.at[pl.ds(s_id * TROW, TROW)],
                        p0.at[pl.ds(s_id * TROW, TROW)])

    @pl.when(c_id == 1)
    def _():
        pltpu.sync_copy(acc.at[pl.ds(s_id * TROW, TROW)],
                        p1.at[pl.ds(s_id * TROW, TROW)])


_deg_kernel = pl.kernel(
    _deg_body,
    out_type=(jax.ShapeDtypeStruct((Np,), jnp.float32),
              jax.ShapeDtypeStruct((Np,), jnp.float32)),
    mesh=_mesh,
    compiler_params=pltpu.CompilerParams(needs_layout_passes=False, use_tc_tiling_on_sc=False),
    scratch_types=[
        pltpu.VMEM_SHARED((Np,), jnp.float32),
        pltpu.VMEM((RI, 128), jnp.int32),
        pltpu.VMEM((RI, 128), jnp.float32),
    ],
)


def _norm_body(disf, rowf, colf, ewf, norm_out,
               dis_v, row_v, col_v, ew_v, nrm_v):
    c_id = lax.axis_index("c")
    s_id = lax.axis_index("s")
    wid = c_id * NS + s_id
    pltpu.sync_copy(disf, dis_v)

    base = wid * TE
    pltpu.sync_copy(rowf.at[pl.ds(base, TE)], row_v)
    pltpu.sync_copy(colf.at[pl.ds(base, TE)], col_v)
    pltpu.sync_copy(ewf.at[pl.ds(base, TE)], ew_v)

    def eloop(i, carry):
        r16 = row_v[pl.ds(i * 16, 16)]
        c16 = col_v[pl.ds(i * 16, 16)]
        a = plsc.load_gather(dis_v, [r16])
        b = plsc.load_gather(dis_v, [c16])
        nrm_v[pl.ds(i * 16, 16)] = -(a * ew_v[pl.ds(i * 16, 16)] * b)
        return carry

    lax.fori_loop(0, TE // 16, eloop, 0)
    pltpu.sync_copy(nrm_v, norm_out.at[pl.ds(base, TE)])


_norm_kernel = pl.kernel(
    _norm_body,
    out_type=jax.ShapeDtypeStruct((Ep,), jnp.float32),
    mesh=_mesh,
    compiler_params=pltpu.CompilerParams(needs_layout_passes=False, use_tc_tiling_on_sc=False),
    scratch_types=[
        pltpu.VMEM((Np,), jnp.float32),
        pltpu.VMEM((TE,), jnp.int32),
        pltpu.VMEM((TE,), jnp.int32),
        pltpu.VMEM((TE,), jnp.float32),
        pltpu.VMEM((TE,), jnp.float32),
    ],
)


def _dis_body(p0_ref, p1_ref, o_ref):
    d = p0_ref[...] + p1_ref[...]
    o_ref[...] = jnp.where(d > 0.0, lax.rsqrt(jnp.where(d > 0.0, d, 1.0)),
                           0.0)


_dis_kernel = pl.pallas_call(
    _dis_body,
    in_specs=[
        pl.BlockSpec((Np // 128, 128), lambda: (0, 0)),
        pl.BlockSpec((Np // 128, 128), lambda: (0, 0)),
    ],
    out_specs=pl.BlockSpec((Np // 128, 128), lambda: (0, 0)),
    out_shape=jax.ShapeDtypeStruct((Np // 128, 128), jnp.float32),
)


def _hop_body(t_hbm, row2d, col2d, nrmf, z48, p0, p1,
              acc, rowi_v, coli_v, nrm_v, rows_v, sem):
    c_id = lax.axis_index("c")
    s_id = lax.axis_index("s")
    wid = c_id * NS + s_id
    pltpu.sync_copy(z48.at[pl.ds(s_id * TROW, TROW)],
                    acc.at[pl.ds(s_id * TROW, TROW)])
    plsc.subcore_barrier()
    base_row = wid * (TE // 128)
    ebase = wid * TE

    def chunk(ci, carry):
        rbase = base_row + ci * RI
        pltpu.sync_copy(row2d.at[pl.ds(rbase, RI)], rowi_v)
        pltpu.sync_copy(col2d.at[pl.ds(rbase, RI)], coli_v)
        pltpu.sync_copy(nrmf.at[pl.ds(ebase + ci * CHUNK, CHUNK)], nrm_v)
        descs = [
            pltpu.async_copy(t_hbm.at[rowi_v.at[j]],
                             rows_v.at[pl.ds(j * 128, 128)], sem)
            for j in range(RI)
        ]
        for d in descs:
            d.wait()

        def scale(g, c2):
            n16 = nrm_v[pl.ds(g * 16, 16)]
            for l in range(16):
                s16 = n16.at[jnp.full((16,), l, jnp.int32)].get(
                    mode="promise_in_bounds")
                e = g * 16 + l
                for j in range(3):
                    rows_v[e, pl.ds(j * 16, 16)] = (
                        rows_v[e, pl.ds(j * 16, 16)] * s16)
            return c2

        lax.fori_loop(0, CHUNK // 16, scale, 0)
        for j in range(RI):
            pltpu.sync_copy(rows_v.at[pl.ds(j * 128, 128)],
                            acc.at[coli_v.at[j]], add=True)
        return carry

    lax.fori_loop(0, NCHUNK, chunk, 0)
    plsc.subcore_barrier()

    @pl.when(c_id == 0)
    def _():
        pltpu.sync_copy(acc.at[pl.ds(s_id * TROW, TROW)],
                        p0.at[pl.ds(s_id * TROW, TROW)])

    @pl.when(c_id == 1)
    def _():
        pltpu.sync_copy(acc.at[pl.ds(s_id * TROW, TROW)],
                        p1.at[pl.ds(s_id * TROW, TROW)])


_hop_kernel = pl.kernel(
    _hop_body,
    out_type=(jax.ShapeDtypeStruct((Np, EMB), jnp.float32),
              jax.ShapeDtypeStruct((Np, EMB), jnp.float32)),
    mesh=_mesh,
    compiler_params=pltpu.CompilerParams(needs_layout_passes=False, use_tc_tiling_on_sc=False),
    scratch_types=[
        pltpu.VMEM_SHARED((Np, EMB), jnp.float32),
        pltpu.VMEM((RI, 128), jnp.int32),
        pltpu.VMEM((RI, 128), jnp.int32),
        pltpu.VMEM((CHUNK,), jnp.float32),
        pltpu.VMEM((CHUNK, EMB), jnp.float32),
        pltpu.SemaphoreType.DMA,
    ],
)


_BLK = 1000
_NBLK = N // _BLK


def _input_body(x_ref, w_ref, b_ref, o_ref):
    h = jnp.dot(x_ref[...], w_ref[...], preferred_element_type=jnp.float32)
    o_ref[...] = jnp.maximum(h + b_ref[...], 0.0)


_input_kernel = pl.pallas_call(
    _input_body,
    grid=(_NBLK,),
    in_specs=[
        pl.BlockSpec((_BLK, D_IN), lambda i: (i, 0)),
        pl.BlockSpec((D_IN, EMB), lambda i: (0, 0)),
        pl.BlockSpec((1, EMB), lambda i: (0, 0)),
    ],
    out_specs=pl.BlockSpec((_BLK, EMB), lambda i: (i, 0)),
    out_shape=jax.ShapeDtypeStruct((N, EMB), jnp.float32),
)


def _comb1_body(p0_ref, p1_ref, o_ref):
    o_ref[...] = p0_ref[...] + p1_ref[...]


_comb1_kernel = pl.pallas_call(
    _comb1_body,
    grid=(_NBLK,),
    in_specs=[
        pl.BlockSpec((_BLK, EMB), lambda i: (i, 0)),
        pl.BlockSpec((_BLK, EMB), lambda i: (i, 0)),
    ],
    out_specs=pl.BlockSpec((_BLK, EMB), lambda i: (i, 0)),
    out_shape=jax.ShapeDtypeStruct((N, EMB), jnp.float32),
)


def _comb_body(p0_ref, p1_ref, tm2_ref, o_ref):
    o_ref[...] = 2.0 * (p0_ref[...] + p1_ref[...]) - tm2_ref[...]


_comb_kernel = pl.pallas_call(
    _comb_body,
    grid=(_NBLK,),
    in_specs=[
        pl.BlockSpec((_BLK, EMB), lambda i: (i, 0)),
        pl.BlockSpec((_BLK, EMB), lambda i: (i, 0)),
        pl.BlockSpec((_BLK, EMB), lambda i: (i, 0)),
    ],
    out_specs=pl.BlockSpec((_BLK, EMB), lambda i: (i, 0)),
    out_shape=jax.ShapeDtypeStruct((N, EMB), jnp.float32),
)


def _final_body(t0, t1, t2, t3, t4, t5, t6, cw, cb, wo, bo, o_ref):
    ts = (t0, t1, t2, t3, t4, t5, t6)
    acc = jnp.dot(ts[0][...], cw[0], preferred_element_type=jnp.float32)
    for k in range(1, K):
        acc = acc + jnp.dot(ts[k][...], cw[k],
                            preferred_element_type=jnp.float32)
    acc = jnp.maximum(acc + cb[...], 0.0)
    o_ref[...] = jnp.dot(acc, wo[...],
                         preferred_element_type=jnp.float32) + bo[...]


_final_kernel = pl.pallas_call(
    _final_body,
    grid=(_NBLK,),
    in_specs=[pl.BlockSpec((_BLK, EMB), lambda i: (i, 0))] * K + [
        pl.BlockSpec((K, EMB, EMB), lambda i: (0, 0, 0)),
        pl.BlockSpec((1, EMB), lambda i: (0, 0)),
        pl.BlockSpec((EMB, D_OUT), lambda i: (0, 0)),
        pl.BlockSpec((1, D_OUT), lambda i: (0, 0)),
    ],
    out_specs=pl.BlockSpec((_BLK, D_OUT), lambda i: (i, 0)),
    out_shape=jax.ShapeDtypeStruct((N, D_OUT), jnp.float32),
)


def kernel(x, edge_index, edge_weight, W_in, b_in, cheb_W, cheb_b, W_out,
           b_out):
    row = edge_index[0].astype(jnp.int32)
    col = edge_index[1].astype(jnp.int32)
    pad = Ep - E
    rowp = jnp.concatenate([row, jnp.zeros((pad,), jnp.int32)])
    colp = jnp.concatenate([col, jnp.zeros((pad,), jnp.int32)])
    ewp = jnp.concatenate([edge_weight, jnp.zeros((pad,), jnp.float32)])
    row2d = rowp.reshape(Ep // 128, 128)
    col2d = colp.reshape(Ep // 128, 128)
    ew2d = ewp.reshape(Ep // 128, 128)
    zn = jnp.zeros((Np,), jnp.float32)
    z48 = jnp.zeros((Np, EMB), jnp.float32)

    dp0, dp1 = _deg_kernel(row2d, ew2d, zn)
    dis = _dis_kernel(dp0.reshape(Np // 128, 128),
                      dp1.reshape(Np // 128, 128)).reshape(Np)
    norm = _norm_kernel(dis, rowp, colp, ewp)
    h = _input_kernel(x, W_in, b_in.reshape(1, EMB))

    ts = [h]
    p0, p1 = _hop_kernel(h, row2d, col2d, norm, z48)
    t1 = _comb1_kernel(p0, p1)
    ts.append(t1)
    tkm2, tkm1 = h, t1
    for _ in range(2, K):
        p0, p1 = _hop_kernel(tkm1, row2d, col2d, norm, z48)
        tk = _comb_kernel(p0, p1, tkm2)
        ts.append(tk)
        tkm2, tkm1 = tkm1, tk

    out = _final_kernel(ts[0], ts[1], ts[2], ts[3], ts[4], ts[5], ts[6],
                        cheb_W, cheb_b.reshape(1, EMB), W_out,
                        b_out.reshape(1, D_OUT))
    return (out, h)


# speedup vs baseline: 9.7431x; 1.1718x over previous
import functools

import jax
import jax.numpy as jnp
from jax import lax
from jax.experimental import pallas as pl
from jax.experimental.pallas import tpu as pltpu
from jax.experimental.pallas import tpu_sc as plsc

N = 10000
E = 320000
D_IN = 128
EMB = 48
K = 7
D_OUT = 10

NC = 2
NS = 16
NW = NC * NS
LANES = 16

Np = 10240
Ep = 327680
TE = Ep // NW
CHUNK = 512
RI = CHUNK // 128
NCHUNK = TE // CHUNK
TROW = Np // NS

_mesh = plsc.VectorSubcoreMesh(core_axis_name="c", subcore_axis_name="s",
                               num_cores=NC, num_subcores=NS)


def _deg_body(row2d, ew2d, zn, p0, p1, acc, idx_v, ew_v):
    c_id = lax.axis_index("c")
    s_id = lax.axis_index("s")
    wid = c_id * NS + s_id
    pltpu.sync_copy(zn.at[pl.ds(s_id * TROW, TROW)],
                    acc.at[pl.ds(s_id * TROW, TROW)])
    plsc.subcore_barrier()
    base_row = wid * (TE // 128)

    def chunk(ci, carry):
        rbase = base_row + ci * RI
        pltpu.sync_copy(row2d.at[pl.ds(rbase, RI)], idx_v)
        pltpu.sync_copy(ew2d.at[pl.ds(rbase, RI)], ew_v)
        for j in range(RI):
            pltpu.sync_copy(ew_v.at[j], acc.at[idx_v.at[j]], add=True)
        return carry

    lax.fori_loop(0, NCHUNK, chunk, 0)
    plsc.subcore_barrier()

    @pl.when(c_id == 0)
    def _():
        pltpu.sync_copy(acc.at[pl.ds(s_id * TROW, TROW)],
                        p0.at[pl.ds(s_id * TROW, TROW)])

    @pl.when(c_id == 1)
    def _():
        pltpu.sync_copy(acc.at[pl.ds(s_id * TROW, TROW)],
                        p1.at[pl.ds(s_id * TROW, TROW)])


_deg_kernel = pl.kernel(
    _deg_body,
    out_type=(jax.ShapeDtypeStruct((Np,), jnp.float32),
              jax.ShapeDtypeStruct((Np,), jnp.float32)),
    mesh=_mesh,
    compiler_params=pltpu.CompilerParams(needs_layout_passes=False, use_tc_tiling_on_sc=False),
    scratch_types=[
        pltpu.VMEM_SHARED((Np,), jnp.float32),
        pltpu.VMEM((RI, 128), jnp.int32),
        pltpu.VMEM((RI, 128), jnp.float32),
    ],
)


def _norm_body(disf, rowf, colf, ewf, norm_out,
               dis_v, row_v, col_v, ew_v, nrm_v):
    c_id = lax.axis_index("c")
    s_id = lax.axis_index("s")
    wid = c_id * NS + s_id
    pltpu.sync_copy(disf, dis_v)

    base = wid * TE
    pltpu.sync_copy(rowf.at[pl.ds(base, TE)], row_v)
    pltpu.sync_copy(colf.at[pl.ds(base, TE)], col_v)
    pltpu.sync_copy(ewf.at[pl.ds(base, TE)], ew_v)

    def eloop(i, carry):
        r16 = row_v[pl.ds(i * 16, 16)]
        c16 = col_v[pl.ds(i * 16, 16)]
        a = plsc.load_gather(dis_v, [r16])
        b = plsc.load_gather(dis_v, [c16])
        nrm_v[pl.ds(i * 16, 16)] = -(a * ew_v[pl.ds(i * 16, 16)] * b)
        return carry

    lax.fori_loop(0, TE // 16, eloop, 0)
    pltpu.sync_copy(nrm_v, norm_out.at[pl.ds(base, TE)])


_norm_kernel = pl.kernel(
    _norm_body,
    out_type=jax.ShapeDtypeStruct((Ep,), jnp.float32),
    mesh=_mesh,
    compiler_params=pltpu.CompilerParams(needs_layout_passes=False, use_tc_tiling_on_sc=False),
    scratch_types=[
        pltpu.VMEM((Np,), jnp.float32),
        pltpu.VMEM((TE,), jnp.int32),
        pltpu.VMEM((TE,), jnp.int32),
        pltpu.VMEM((TE,), jnp.float32),
        pltpu.VMEM((TE,), jnp.float32),
    ],
)


def _dis_body(p0_ref, p1_ref, o_ref):
    d = p0_ref[...] + p1_ref[...]
    o_ref[...] = jnp.where(d > 0.0, lax.rsqrt(jnp.where(d > 0.0, d, 1.0)),
                           0.0)


_dis_kernel = pl.pallas_call(
    _dis_body,
    in_specs=[
        pl.BlockSpec((Np // 128, 128), lambda: (0, 0)),
        pl.BlockSpec((Np // 128, 128), lambda: (0, 0)),
    ],
    out_specs=pl.BlockSpec((Np // 128, 128), lambda: (0, 0)),
    out_shape=jax.ShapeDtypeStruct((Np // 128, 128), jnp.float32),
)


NBUF = 2
NROUND = NCHUNK // NBUF


def _hop_body(t_hbm, row2d, col2d, nrmf, z48, p0, p1,
              acc, rowi_v, coli_v, nrm_v, rows0, rows1, sem_g, sem_s):
    c_id = lax.axis_index("c")
    s_id = lax.axis_index("s")
    wid = c_id * NS + s_id
    pltpu.sync_copy(z48.at[pl.ds(s_id * TROW, TROW)],
                    acc.at[pl.ds(s_id * TROW, TROW)])
    base_row = wid * (TE // 128)
    pltpu.sync_copy(row2d.at[pl.ds(base_row, TE // 128)], rowi_v)
    pltpu.sync_copy(col2d.at[pl.ds(base_row, TE // 128)], coli_v)
    pltpu.sync_copy(nrmf.at[pl.ds(wid * TE, TE)], nrm_v)
    plsc.subcore_barrier()
    rows = (rows0, rows1)

    def fire_gather(c, buf):
        for j in range(RI):
            pltpu.async_copy(t_hbm.at[rowi_v.at[c * RI + j]],
                             buf.at[pl.ds(j * 128, 128)], sem_g)

    def wait_gather(c, buf):
        for j in range(RI):
            pltpu.make_async_copy(t_hbm.at[rowi_v.at[c * RI + j]],
                                  buf.at[pl.ds(j * 128, 128)], sem_g).wait()

    def fire_scatter(c, buf):
        for j in range(RI):
            pltpu.async_copy(buf.at[pl.ds(j * 128, 128)],
                             acc.at[coli_v.at[c * RI + j]], sem_s, add=True)

    def wait_scatter(c, buf):
        for j in range(RI):
            pltpu.make_async_copy(buf.at[pl.ds(j * 128, 128)],
                                  acc.at[coli_v.at[c * RI + j]],
                                  sem_s).wait()

    def scale(c, buf):
        def sbody(g, c2):
            n16 = nrm_v[pl.ds(c * CHUNK + g * 16, 16)]
            for l in range(16):
                s16 = n16.at[jnp.full((16,), l, jnp.int32)].get(
                    mode="promise_in_bounds")
                e = g * 16 + l
                for j in range(3):
                    buf[e, pl.ds(j * 16, 16)] = buf[e, pl.ds(j * 16, 16)] * s16
            return c2

        lax.fori_loop(0, CHUNK // 16, sbody, 0)

    for b in range(NBUF):
        fire_gather(b, rows[b])

    def round_body(g, carry):
        for b in range(NBUF):
            c = g * NBUF + b
            wait_gather(c, rows[b])
            scale(c, rows[b])
            fire_scatter(c, rows[b])

        @pl.when(g < NROUND - 1)
        def _():
            for b in range(NBUF):
                c = g * NBUF + b
                wait_scatter(c, rows[b])
                fire_gather(c + NBUF, rows[b])

        return carry

    lax.fori_loop(0, NROUND, round_body, 0)
    for b in range(NBUF):
        wait_scatter(0, rows[b])
    plsc.subcore_barrier()

    @pl.when(c_id == 0)
    def _():
        pltpu.sync_copy(acc.at[pl.ds(s_id * TROW, TROW)],
                        p0.at[pl.ds(s_id * TROW, TROW)])

    @pl.when(c_id == 1)
    def _():
        pltpu.sync_copy(acc.at[pl.ds(s_id * TROW, TROW)],
                        p1.at[pl.ds(s_id * TROW, TROW)])


_hop_kernel = pl.kernel(
    _hop_body,
    out_type=(jax.ShapeDtypeStruct((Np, EMB), jnp.float32),
              jax.ShapeDtypeStruct((Np, EMB), jnp.float32)),
    mesh=_mesh,
    compiler_params=pltpu.CompilerParams(needs_layout_passes=False, use_tc_tiling_on_sc=False),
    scratch_types=[
        pltpu.VMEM_SHARED((Np, EMB), jnp.float32),
        pltpu.VMEM((TE // 128, 128), jnp.int32),
        pltpu.VMEM((TE // 128, 128), jnp.int32),
        pltpu.VMEM((TE,), jnp.float32),
        pltpu.VMEM((CHUNK, EMB), jnp.float32),
        pltpu.VMEM((CHUNK, EMB), jnp.float32),
        pltpu.SemaphoreType.DMA,
        pltpu.SemaphoreType.DMA,
    ],
)


_BLK = 1000
_NBLK = N // _BLK


def _input_body(x_ref, w_ref, b_ref, o_ref):
    h = jnp.dot(x_ref[...], w_ref[...], preferred_element_type=jnp.float32)
    o_ref[...] = jnp.maximum(h + b_ref[...], 0.0)


_input_kernel = pl.pallas_call(
    _input_body,
    grid=(_NBLK,),
    in_specs=[
        pl.BlockSpec((_BLK, D_IN), lambda i: (i, 0)),
        pl.BlockSpec((D_IN, EMB), lambda i: (0, 0)),
        pl.BlockSpec((1, EMB), lambda i: (0, 0)),
    ],
    out_specs=pl.BlockSpec((_BLK, EMB), lambda i: (i, 0)),
    out_shape=jax.ShapeDtypeStruct((N, EMB), jnp.float32),
)


def _comb1_body(p0_ref, p1_ref, o_ref):
    o_ref[...] = p0_ref[...] + p1_ref[...]


_comb1_kernel = pl.pallas_call(
    _comb1_body,
    grid=(_NBLK,),
    in_specs=[
        pl.BlockSpec((_BLK, EMB), lambda i: (i, 0)),
        pl.BlockSpec((_BLK, EMB), lambda i: (i, 0)),
    ],
    out_specs=pl.BlockSpec((_BLK, EMB), lambda i: (i, 0)),
    out_shape=jax.ShapeDtypeStruct((N, EMB), jnp.float32),
)


def _comb_body(p0_ref, p1_ref, tm2_ref, o_ref):
    o_ref[...] = 2.0 * (p0_ref[...] + p1_ref[...]) - tm2_ref[...]


_comb_kernel = pl.pallas_call(
    _comb_body,
    grid=(_NBLK,),
    in_specs=[
        pl.BlockSpec((_BLK, EMB), lambda i: (i, 0)),
        pl.BlockSpec((_BLK, EMB), lambda i: (i, 0)),
        pl.BlockSpec((_BLK, EMB), lambda i: (i, 0)),
    ],
    out_specs=pl.BlockSpec((_BLK, EMB), lambda i: (i, 0)),
    out_shape=jax.ShapeDtypeStruct((N, EMB), jnp.float32),
)


def _final_body(t0, t1, t2, t3, t4, t5, t6, cw, cb, wo, bo, o_ref):
    ts = (t0, t1, t2, t3, t4, t5, t6)
    acc = jnp.dot(ts[0][...], cw[0], preferred_element_type=jnp.float32)
    for k in range(1, K):
        acc = acc + jnp.dot(ts[k][...], cw[k],
                            preferred_element_type=jnp.float32)
    acc = jnp.maximum(acc + cb[...], 0.0)
    o_ref[...] = jnp.dot(acc, wo[...],
                         preferred_element_type=jnp.float32) + bo[...]


_final_kernel = pl.pallas_call(
    _final_body,
    grid=(_NBLK,),
    in_specs=[pl.BlockSpec((_BLK, EMB), lambda i: (i, 0))] * K + [
        pl.BlockSpec((K, EMB, EMB), lambda i: (0, 0, 0)),
        pl.BlockSpec((1, EMB), lambda i: (0, 0)),
        pl.BlockSpec((EMB, D_OUT), lambda i: (0, 0)),
        pl.BlockSpec((1, D_OUT), lambda i: (0, 0)),
    ],
    out_specs=pl.BlockSpec((_BLK, D_OUT), lambda i: (i, 0)),
    out_shape=jax.ShapeDtypeStruct((N, D_OUT), jnp.float32),
)


def kernel(x, edge_index, edge_weight, W_in, b_in, cheb_W, cheb_b, W_out,
           b_out):
    row = edge_index[0].astype(jnp.int32)
    col = edge_index[1].astype(jnp.int32)
    pad = Ep - E
    rowp = jnp.concatenate([row, jnp.zeros((pad,), jnp.int32)])
    colp = jnp.concatenate([col, jnp.zeros((pad,), jnp.int32)])
    ewp = jnp.concatenate([edge_weight, jnp.zeros((pad,), jnp.float32)])
    row2d = rowp.reshape(Ep // 128, 128)
    col2d = colp.reshape(Ep // 128, 128)
    ew2d = ewp.reshape(Ep // 128, 128)
    zn = jnp.zeros((Np,), jnp.float32)
    z48 = jnp.zeros((Np, EMB), jnp.float32)

    dp0, dp1 = _deg_kernel(row2d, ew2d, zn)
    dis = _dis_kernel(dp0.reshape(Np // 128, 128),
                      dp1.reshape(Np // 128, 128)).reshape(Np)
    norm = _norm_kernel(dis, rowp, colp, ewp)
    h = _input_kernel(x, W_in, b_in.reshape(1, EMB))

    ts = [h]
    p0, p1 = _hop_kernel(h, row2d, col2d, norm, z48)
    t1 = _comb1_kernel(p0, p1)
    ts.append(t1)
    tkm2, tkm1 = h, t1
    for _ in range(2, K):
        p0, p1 = _hop_kernel(tkm1, row2d, col2d, norm, z48)
        tk = _comb_kernel(p0, p1, tkm2)
        ts.append(tk)
        tkm2, tkm1 = tkm1, tk

    out = _final_kernel(ts[0], ts[1], ts[2], ts[3], ts[4], ts[5], ts[6],
                        cheb_W, cheb_b.reshape(1, EMB), W_out,
                        b_out.reshape(1, D_OUT))
    return (out, h)


# speedup vs baseline: 10.1326x; 1.0400x over previous
import functools

import jax
import jax.numpy as jnp
from jax import lax
from jax.experimental import pallas as pl
from jax.experimental.pallas import tpu as pltpu
from jax.experimental.pallas import tpu_sc as plsc

N = 10000
E = 320000
D_IN = 128
EMB = 48
K = 7
D_OUT = 10

NC = 2
NS = 16
NW = NC * NS
LANES = 16

Np = 10240
Ep = 327680
TE = Ep // NW
CHUNK = 512
RI = CHUNK // 128
NCHUNK = TE // CHUNK
TROW = Np // NS

_mesh = plsc.VectorSubcoreMesh(core_axis_name="c", subcore_axis_name="s",
                               num_cores=NC, num_subcores=NS)


def _deg_body(row2d, ew2d, zn, p0, p1, acc, idx_v, ew_v):
    c_id = lax.axis_index("c")
    s_id = lax.axis_index("s")
    wid = c_id * NS + s_id
    pltpu.sync_copy(zn.at[pl.ds(s_id * TROW, TROW)],
                    acc.at[pl.ds(s_id * TROW, TROW)])
    plsc.subcore_barrier()
    base_row = wid * (TE // 128)

    def chunk(ci, carry):
        rbase = base_row + ci * RI
        pltpu.sync_copy(row2d.at[pl.ds(rbase, RI)], idx_v)
        pltpu.sync_copy(ew2d.at[pl.ds(rbase, RI)], ew_v)
        for j in range(RI):
            pltpu.sync_copy(ew_v.at[j], acc.at[idx_v.at[j]], add=True)
        return carry

    lax.fori_loop(0, NCHUNK, chunk, 0)
    plsc.subcore_barrier()

    @pl.when(c_id == 0)
    def _():
        pltpu.sync_copy(acc.at[pl.ds(s_id * TROW, TROW)],
                        p0.at[pl.ds(s_id * TROW, TROW)])

    @pl.when(c_id == 1)
    def _():
        pltpu.sync_copy(acc.at[pl.ds(s_id * TROW, TROW)],
                        p1.at[pl.ds(s_id * TROW, TROW)])


_deg_kernel = pl.kernel(
    _deg_body,
    out_type=(jax.ShapeDtypeStruct((Np,), jnp.float32),
              jax.ShapeDtypeStruct((Np,), jnp.float32)),
    mesh=_mesh,
    compiler_params=pltpu.CompilerParams(needs_layout_passes=False, use_tc_tiling_on_sc=False),
    scratch_types=[
        pltpu.VMEM_SHARED((Np,), jnp.float32),
        pltpu.VMEM((RI, 128), jnp.int32),
        pltpu.VMEM((RI, 128), jnp.float32),
    ],
)


def _norm_body(disf, rowf, colf, ewf, norm_out,
               dis_v, row_v, col_v, ew_v, nrm_v):
    c_id = lax.axis_index("c")
    s_id = lax.axis_index("s")
    wid = c_id * NS + s_id
    pltpu.sync_copy(disf, dis_v)

    base = wid * TE
    pltpu.sync_copy(rowf.at[pl.ds(base, TE)], row_v)
    pltpu.sync_copy(colf.at[pl.ds(base, TE)], col_v)
    pltpu.sync_copy(ewf.at[pl.ds(base, TE)], ew_v)

    def eloop(i, carry):
        r16 = row_v[pl.ds(i * 16, 16)]
        c16 = col_v[pl.ds(i * 16, 16)]
        a = plsc.load_gather(dis_v, [r16])
        b = plsc.load_gather(dis_v, [c16])
        nrm_v[pl.ds(i * 16, 16)] = -(a * ew_v[pl.ds(i * 16, 16)] * b)
        return carry

    lax.fori_loop(0, TE // 16, eloop, 0)
    pltpu.sync_copy(nrm_v, norm_out.at[pl.ds(base, TE)])


_norm_kernel = pl.kernel(
    _norm_body,
    out_type=jax.ShapeDtypeStruct((Ep,), jnp.float32),
    mesh=_mesh,
    compiler_params=pltpu.CompilerParams(needs_layout_passes=False, use_tc_tiling_on_sc=False),
    scratch_types=[
        pltpu.VMEM((Np,), jnp.float32),
        pltpu.VMEM((TE,), jnp.int32),
        pltpu.VMEM((TE,), jnp.int32),
        pltpu.VMEM((TE,), jnp.float32),
        pltpu.VMEM((TE,), jnp.float32),
    ],
)


def _dis_body(p0_ref, p1_ref, o_ref):
    d = p0_ref[...] + p1_ref[...]
    o_ref[...] = jnp.where(d > 0.0, lax.rsqrt(jnp.where(d > 0.0, d, 1.0)),
                           0.0)


_dis_kernel = pl.pallas_call(
    _dis_body,
    in_specs=[
        pl.BlockSpec((Np // 128, 128), lambda: (0, 0)),
        pl.BlockSpec((Np // 128, 128), lambda: (0, 0)),
    ],
    out_specs=pl.BlockSpec((Np // 128, 128), lambda: (0, 0)),
    out_shape=jax.ShapeDtypeStruct((Np // 128, 128), jnp.float32),
)


NBUF = 2
NROUND = NCHUNK // NBUF


def _hop_body(t_hbm, row2d, col2d, nrmf, z48, p0, p1,
              acc, rowi_v, coli_v, nrm_v, rows0, rows1, sem_g, sem_s):
    c_id = lax.axis_index("c")
    s_id = lax.axis_index("s")
    wid = c_id * NS + s_id
    pltpu.sync_copy(z48.at[pl.ds(s_id * TROW, TROW)],
                    acc.at[pl.ds(s_id * TROW, TROW)])
    base_row = wid * (TE // 128)
    pltpu.sync_copy(row2d.at[pl.ds(base_row, TE // 128)], rowi_v)
    pltpu.sync_copy(col2d.at[pl.ds(base_row, TE // 128)], coli_v)
    pltpu.sync_copy(nrmf.at[pl.ds(wid * TE, TE)], nrm_v)
    plsc.subcore_barrier()
    rows = (rows0, rows1)

    def fire_gather(c, buf):
        for j in range(RI):
            pltpu.async_copy(t_hbm.at[rowi_v.at[c * RI + j]],
                             buf.at[pl.ds(j * 128, 128)], sem_g)

    def wait_gather(c, buf):
        for j in range(RI):
            pltpu.make_async_copy(t_hbm.at[rowi_v.at[c * RI + j]],
                                  buf.at[pl.ds(j * 128, 128)], sem_g).wait()

    def fire_scatter(c, buf):
        for j in range(RI):
            pltpu.async_copy(buf.at[pl.ds(j * 128, 128)],
                             acc.at[coli_v.at[c * RI + j]], sem_s, add=True)

    def wait_scatter(c, buf):
        for j in range(RI):
            pltpu.make_async_copy(buf.at[pl.ds(j * 128, 128)],
                                  acc.at[coli_v.at[c * RI + j]],
                                  sem_s).wait()

    def scale(c, buf):
        def sbody(g, c2):
            n16 = nrm_v[pl.ds(c * CHUNK + g * 16, 16)]
            for l in range(16):
                s16 = n16.at[jnp.full((16,), l, jnp.int32)].get(
                    mode="promise_in_bounds")
                e = g * 16 + l
                for j in range(3):
                    buf[e, pl.ds(j * 16, 16)] = buf[e, pl.ds(j * 16, 16)] * s16
            return c2

        lax.fori_loop(0, CHUNK // 16, sbody, 0)

    for b in range(NBUF):
        fire_gather(b, rows[b])

    def round_body(g, carry):
        for b in range(NBUF):
            c = g * NBUF + b
            wait_gather(c, rows[b])
            scale(c, rows[b])
            fire_scatter(c, rows[b])

        @pl.when(g < NROUND - 1)
        def _():
            for b in range(NBUF):
                c = g * NBUF + b
                wait_scatter(c, rows[b])
                fire_gather(c + NBUF, rows[b])

        return carry

    lax.fori_loop(0, NROUND, round_body, 0)
    for b in range(NBUF):
        wait_scatter(0, rows[b])
    plsc.subcore_barrier()

    @pl.when(c_id == 0)
    def _():
        pltpu.sync_copy(acc.at[pl.ds(s_id * TROW, TROW)],
                        p0.at[pl.ds(s_id * TROW, TROW)])

    @pl.when(c_id == 1)
    def _():
        pltpu.sync_copy(acc.at[pl.ds(s_id * TROW, TROW)],
                        p1.at[pl.ds(s_id * TROW, TROW)])


_hop_kernel = pl.kernel(
    _hop_body,
    out_type=(jax.ShapeDtypeStruct((Np, EMB), jnp.float32),
              jax.ShapeDtypeStruct((Np, EMB), jnp.float32)),
    mesh=_mesh,
    compiler_params=pltpu.CompilerParams(needs_layout_passes=False, use_tc_tiling_on_sc=False),
    scratch_types=[
        pltpu.VMEM_SHARED((Np, EMB), jnp.float32),
        pltpu.VMEM((TE // 128, 128), jnp.int32),
        pltpu.VMEM((TE // 128, 128), jnp.int32),
        pltpu.VMEM((TE,), jnp.float32),
        pltpu.VMEM((CHUNK, EMB), jnp.float32),
        pltpu.VMEM((CHUNK, EMB), jnp.float32),
        pltpu.SemaphoreType.DMA,
        pltpu.SemaphoreType.DMA,
    ],
)


_BLK = 1000
_NBLK = N // _BLK


def _input_body(x_ref, w_ref, b_ref, o_ref):
    h = jnp.dot(x_ref[...], w_ref[...], preferred_element_type=jnp.float32)
    o_ref[...] = jnp.maximum(h + b_ref[...], 0.0)


_input_kernel = pl.pallas_call(
    _input_body,
    grid=(_NBLK,),
    in_specs=[
        pl.BlockSpec((_BLK, D_IN), lambda i: (i, 0)),
        pl.BlockSpec((D_IN, EMB), lambda i: (0, 0)),
        pl.BlockSpec((1, EMB), lambda i: (0, 0)),
    ],
    out_specs=pl.BlockSpec((_BLK, EMB), lambda i: (i, 0)),
    out_shape=jax.ShapeDtypeStruct((N, EMB), jnp.float32),
)


def _comb1_body(p0_ref, p1_ref, o_ref):
    o_ref[...] = p0_ref[...] + p1_ref[...]


_comb1_kernel = pl.pallas_call(
    _comb1_body,
    grid=(_NBLK,),
    in_specs=[
        pl.BlockSpec((_BLK, EMB), lambda i: (i, 0)),
        pl.BlockSpec((_BLK, EMB), lambda i: (i, 0)),
    ],
    out_specs=pl.BlockSpec((_BLK, EMB), lambda i: (i, 0)),
    out_shape=jax.ShapeDtypeStruct((N, EMB), jnp.float32),
)


def _comb_body(p0_ref, p1_ref, tm2_ref, o_ref):
    o_ref[...] = 2.0 * (p0_ref[...] + p1_ref[...]) - tm2_ref[...]


_comb_kernel = pl.pallas_call(
    _comb_body,
    grid=(_NBLK,),
    in_specs=[
        pl.BlockSpec((_BLK, EMB), lambda i: (i, 0)),
        pl.BlockSpec((_BLK, EMB), lambda i: (i, 0)),
        pl.BlockSpec((_BLK, EMB), lambda i: (i, 0)),
    ],
    out_specs=pl.BlockSpec((_BLK, EMB), lambda i: (i, 0)),
    out_shape=jax.ShapeDtypeStruct((N, EMB), jnp.float32),
)


def _final_body(t0, t1, t2, t3, t4, t5, t6, cw, cb, wo, bo, o_ref):
    ts = (t0, t1, t2, t3, t4, t5, t6)
    acc = jnp.dot(ts[0][...], cw[0], preferred_element_type=jnp.float32)
    for k in range(1, K):
        acc = acc + jnp.dot(ts[k][...], cw[k],
                            preferred_element_type=jnp.float32)
    acc = jnp.maximum(acc + cb[...], 0.0)
    o_ref[...] = jnp.dot(acc, wo[...],
                         preferred_element_type=jnp.float32) + bo[...]


_final_kernel = pl.pallas_call(
    _final_body,
    grid=(_NBLK,),
    in_specs=[pl.BlockSpec((_BLK, EMB), lambda i: (i, 0))] * K + [
        pl.BlockSpec((K, EMB, EMB), lambda i: (0, 0, 0)),
        pl.BlockSpec((1, EMB), lambda i: (0, 0)),
        pl.BlockSpec((EMB, D_OUT), lambda i: (0, 0)),
        pl.BlockSpec((1, D_OUT), lambda i: (0, 0)),
    ],
    out_specs=pl.BlockSpec((_BLK, D_OUT), lambda i: (i, 0)),
    out_shape=jax.ShapeDtypeStruct((N, D_OUT), jnp.float32),
)


def kernel(x, edge_index, edge_weight, W_in, b_in, cheb_W, cheb_b, W_out,
           b_out):
    row = edge_index[0].astype(jnp.int32)
    col = edge_index[1].astype(jnp.int32)
    pad = Ep - E
    pad_idx = jnp.arange(pad, dtype=jnp.int32) % Np
    rowp = jnp.concatenate([row, jnp.zeros((pad,), jnp.int32)])
    colp = jnp.concatenate([col, pad_idx])
    rowp_deg = jnp.concatenate([row, pad_idx])
    ewp = jnp.concatenate([edge_weight, jnp.zeros((pad,), jnp.float32)])
    row2d = rowp.reshape(Ep // 128, 128)
    col2d = colp.reshape(Ep // 128, 128)
    rowdeg2d = rowp_deg.reshape(Ep // 128, 128)
    ew2d = ewp.reshape(Ep // 128, 128)
    zn = jnp.zeros((Np,), jnp.float32)
    z48 = jnp.zeros((Np, EMB), jnp.float32)

    dp0, dp1 = _deg_kernel(rowdeg2d, ew2d, zn)
    dis = _dis_kernel(dp0.reshape(Np // 128, 128),
                      dp1.reshape(Np // 128, 128)).reshape(Np)
    norm = _norm_kernel(dis, rowp, colp, ewp)
    h = _input_kernel(x, W_in, b_in.reshape(1, EMB))

    ts = [h]
    p0, p1 = _hop_kernel(h, row2d, col2d, norm, z48)
    t1 = _comb1_kernel(p0, p1)
    ts.append(t1)
    tkm2, tkm1 = h, t1
    for _ in range(2, K):
        p0, p1 = _hop_kernel(tkm1, row2d, col2d, norm, z48)
        tk = _comb_kernel(p0, p1, tkm2)
        ts.append(tk)
        tkm2, tkm1 = tkm1, tk

    out = _final_kernel(ts[0], ts[1], ts[2], ts[3], ts[4], ts[5], ts[6],
                        cheb_W, cheb_b.reshape(1, EMB), W_out,
                        b_out.reshape(1, D_OUT))
    return (out, h)
